# baseline (speedup 1.0000x reference)
import functools

import jax
import jax.numpy as jnp
from jax import lax
from jax.experimental import pallas as pl
from jax.experimental.pallas import tpu as pltpu, tpu_sc as plsc

NUM_CORES = 2
NUM_SUBCORES = 16
NUM_WORKERS = NUM_CORES * NUM_SUBCORES

VOCAB = 1_000_000
DIM = 64
TOTAL = 4096 * 200
PER_WORKER = TOTAL // NUM_WORKERS
CHUNK = 128
N_CHUNKS = PER_WORKER // CHUNK
NBUF = 4

_mesh = plsc.VectorSubcoreMesh(
    core_axis_name="c",
    subcore_axis_name="s",
    num_cores=NUM_CORES,
    num_subcores=NUM_SUBCORES,
)


@functools.partial(
    pl.kernel,
    out_type=jax.ShapeDtypeStruct((NUM_WORKERS, N_CHUNKS, CHUNK, DIM), jnp.float32),
    mesh=_mesh,
    scratch_types=[
        pltpu.VMEM((N_CHUNKS, CHUNK), jnp.int32),
        pltpu.VMEM((NBUF, CHUNK, DIM), jnp.float32),
        pltpu.SemaphoreType.DMA,
        pltpu.SemaphoreType.DMA,
    ],
    compiler_params=pltpu.CompilerParams(use_tc_tiling_on_sc=False),
)
def _embed_sc(table_hbm, ids_hbm, out_hbm, idx_v, rows_v, sem_g, sem_s):
    wid = lax.axis_index("s") * NUM_CORES + lax.axis_index("c")

    pltpu.sync_copy(ids_hbm.at[wid], idx_v)

    def fire_gather(j, b):
        pltpu.async_copy(table_hbm.at[idx_v.at[j]], rows_v.at[b], sem_g)

    def fire_store(j, b):
        pltpu.async_copy(rows_v.at[b], out_hbm.at[wid, j], sem_s)

    def wait_gather():
        pltpu.make_async_copy(table_hbm.at[idx_v.at[0]], rows_v.at[0], sem_g).wait()

    def wait_store():
        pltpu.make_async_copy(rows_v.at[0], out_hbm.at[wid, 0], sem_s).wait()

    for b in range(NBUF - 1):
        fire_gather(b, b)

    for t in range(NBUF):
        if t >= 1:
            wait_store()
        fire_gather(t + NBUF - 1, (t + NBUF - 1) % NBUF)
        wait_gather()
        fire_store(t, t)

    @pl.loop(1, N_CHUNKS // NBUF - 1)
    def _block(k):
        j0 = k * NBUF
        for t in range(NBUF):
            wait_store()
            fire_gather(j0 + t + NBUF - 1, (t + NBUF - 1) % NBUF)
            wait_gather()
            fire_store(j0 + t, t)

    j0 = N_CHUNKS - NBUF
    for t in range(NBUF):
        wait_store()
        if t == 0:
            fire_gather(N_CHUNKS - 1, (N_CHUNKS - 1) % NBUF)
        wait_gather()
        fire_store(j0 + t, t)

    wait_store()


def kernel(input_ids, weight):
    ids = input_ids.astype(jnp.int32).reshape(NUM_WORKERS, N_CHUNKS, CHUNK)
    out = _embed_sc(weight, ids)
    return out.reshape(input_ids.shape[0], input_ids.shape[1], DIM)


# speedup vs baseline: 1.0019x; 1.0019x over previous
import functools

import jax
import jax.numpy as jnp
from jax import lax
from jax.experimental import pallas as pl
from jax.experimental.pallas import tpu as pltpu, tpu_sc as plsc

NUM_CORES = 2
NUM_SUBCORES = 16
NUM_WORKERS = NUM_CORES * NUM_SUBCORES

DIM = 64
TOTAL = 4096 * 200
PER_WORKER = TOTAL // NUM_WORKERS

CHUNK = 512
NBUF = 2
GDEPTH = 1
N_CHUNKS = PER_WORKER // CHUNK

_mesh = plsc.VectorSubcoreMesh(
    core_axis_name="c",
    subcore_axis_name="s",
    num_cores=NUM_CORES,
    num_subcores=NUM_SUBCORES,
)


@functools.partial(
    pl.kernel,
    out_type=jax.ShapeDtypeStruct((NUM_WORKERS, N_CHUNKS, CHUNK, DIM), jnp.float32),
    mesh=_mesh,
    scratch_types=[
        pltpu.VMEM((N_CHUNKS, CHUNK), jnp.int32),
        pltpu.VMEM((NBUF, CHUNK, DIM), jnp.float32),
        pltpu.SemaphoreType.DMA,
        pltpu.SemaphoreType.DMA,
    ],
    compiler_params=pltpu.CompilerParams(use_tc_tiling_on_sc=False),
)
def _embed_sc(table_hbm, ids_hbm, out_hbm, idx_v, rows_v, sem_g, sem_s):
    wid = lax.axis_index("s") * NUM_CORES + lax.axis_index("c")

    pltpu.sync_copy(ids_hbm.at[wid], idx_v)

    def fire_gather(j, b):
        pltpu.async_copy(table_hbm.at[idx_v.at[j]], rows_v.at[b], sem_g)

    def fire_store(j, b):
        pltpu.async_copy(rows_v.at[b], out_hbm.at[wid, j], sem_s)

    def wait_gather():
        pltpu.make_async_copy(table_hbm.at[idx_v.at[0]], rows_v.at[0], sem_g).wait()

    def wait_store():
        pltpu.make_async_copy(rows_v.at[0], out_hbm.at[wid, 0], sem_s).wait()

    for b in range(GDEPTH):
        fire_gather(b, b)

    for t in range(NBUF):
        if t >= NBUF - GDEPTH:
            wait_store()
        fire_gather(t + GDEPTH, (t + GDEPTH) % NBUF)
        wait_gather()
        fire_store(t, t)

    @pl.loop(1, N_CHUNKS // NBUF - 1)
    def _block(k):
        j0 = k * NBUF
        for t in range(NBUF):
            wait_store()
            fire_gather(j0 + t + GDEPTH, (t + GDEPTH) % NBUF)
            wait_gather()
            fire_store(j0 + t, t)

    j0 = N_CHUNKS - NBUF
    for t in range(NBUF):
        if t < NBUF - GDEPTH:
            wait_store()
            fire_gather(j0 + t + GDEPTH, (t + GDEPTH) % NBUF)
        wait_gather()
        fire_store(j0 + t, t)

    for _ in range(NBUF):
        wait_store()


def kernel(input_ids, weight):
    ids = input_ids.astype(jnp.int32).reshape(NUM_WORKERS, N_CHUNKS, CHUNK)
    out = _embed_sc(weight, ids)
    return out.reshape(input_ids.shape[0], input_ids.shape[1], DIM)
